# Initial kernel scaffold; baseline (speedup 1.0000x reference)
import jax
import jax.numpy as jnp
from jax.experimental import pallas as pl


def kernel(atom_feats, atom_mask, molecule_atom_lens, W):
    raise NotImplementedError("write your pallas kernel here")


# speedup vs baseline: 3.4756x; 3.4756x over previous
import jax
import jax.numpy as jnp
from jax import lax
from jax.experimental import pallas as pl
from jax.experimental.pallas import tpu as pltpu
from jax.experimental.pallas import tpu_sc as plsc

B, M, N, DA, DS = 16, 4096, 1024, 128, 128
R = B * M
T = B * N
NC, NS = 2, 16
NW = NC * NS
TPW = T // NW
CT = 32
NCHUNK = TPW // CT
PBLK = 4096


def _proj_body(a_ref, w_ref, x_ref):
    x_ref[...] = jnp.maximum(
        lax.dot_general(a_ref[...], w_ref[...], (((1,), (1,)), ((), ())),
                        preferred_element_type=jnp.float32),
        0.0)


def _index_body(lens_ref, idx_ref, w_ref):
    lens = lens_ref[...]
    lens_f = lens.astype(jnp.float32)
    k = lax.broadcasted_iota(jnp.int32, (N, N), 0)
    n = lax.broadcasted_iota(jnp.int32, (N, N), 1)
    tri = (k <= n).astype(jnp.float32)
    ends = lax.dot_general(lens_f, tri, (((1,), (0,)), ((), ())),
                           preferred_element_type=jnp.float32)
    starts = ends.astype(jnp.int32) - lens
    base = starts + lax.broadcasted_iota(jnp.int32, (B, N), 0) * M
    inv = 1.0 / jnp.maximum(lens_f, 1.0)
    for j in range(4):
        idx_ref[j] = jnp.minimum(base + j, R - 1)
        w_ref[j] = jnp.where(lens > j, inv, 0.0)


def _sc_pool_body(x_hbm, idx_hbm, w_hbm, out_hbm,
                  idx_v, w_v, rows0, rows1, out_v, sem0, sem1):
    wid = lax.axis_index("s") * NC + lax.axis_index("c")
    row0 = wid * NCHUNK
    pltpu.sync_copy(idx_hbm.at[pl.ds(row0, NCHUNK)], idx_v)
    pltpu.sync_copy(w_hbm.at[pl.ds(row0, NCHUNK)], w_v)
    bufs = (rows0, rows1)
    sems = (sem0, sem1)
    handles = [None, None]
    handles[0] = pltpu.async_copy(x_hbm.at[idx_v.at[0]], bufs[0], sems[0])
    for c in range(NCHUNK):
        handles[c % 2].wait()
        if c + 1 < NCHUNK:
            handles[(c + 1) % 2] = pltpu.async_copy(
                x_hbm.at[idx_v.at[c + 1]], bufs[(c + 1) % 2], sems[(c + 1) % 2])
        rows = bufs[c % 2]

        def group_body(tg, carry, rows=rows, c=c):
            wvec = w_v[c, pl.ds(tg * 16, 16)]
            for u in range(4):
                q = tg * 16 + u * 4
                t = tg * 4 + u
                w0 = wvec[u * 4]
                w1 = wvec[u * 4 + 1]
                w2 = wvec[u * 4 + 2]
                w3 = wvec[u * 4 + 3]
                for g in range(8):
                    s = pl.ds(g * 16, 16)
                    acc = (rows[q, s] * w0 + rows[q + 1, s] * w1
                           + rows[q + 2, s] * w2 + rows[q + 3, s] * w3)
                    out_v[t, s] = acc
            return carry

        lax.fori_loop(0, CT // 4, group_body, 0)
        pltpu.sync_copy(out_v, out_hbm.at[pl.ds(wid * TPW + c * CT, CT)])


def kernel(atom_feats, atom_mask, molecule_atom_lens, W):
    del atom_mask
    a2 = atom_feats.reshape(R, DA)
    x = pl.pallas_call(
        _proj_body,
        grid=(R // PBLK,),
        in_specs=[pl.BlockSpec((PBLK, DA), lambda i: (i, 0)),
                  pl.BlockSpec((DS, DA), lambda i: (0, 0))],
        out_specs=pl.BlockSpec((PBLK, DS), lambda i: (i, 0)),
        out_shape=jax.ShapeDtypeStruct((R, DS), jnp.float32),
    )(a2, W)

    idx4, w4 = pl.pallas_call(
        _index_body,
        out_shape=(jax.ShapeDtypeStruct((4, B, N), jnp.int32),
                   jax.ShapeDtypeStruct((4, B, N), jnp.float32)),
    )(molecule_atom_lens)
    idx = jnp.transpose(idx4, (1, 2, 0)).reshape(T * 4 // 128, 128)
    wts = jnp.transpose(w4, (1, 2, 0)).reshape(T * 4 // 128, 128)

    pool = pl.kernel(
        _sc_pool_body,
        out_type=jax.ShapeDtypeStruct((T, DS), jnp.float32),
        mesh=plsc.VectorSubcoreMesh(core_axis_name="c", subcore_axis_name="s"),
        scratch_types=[
            pltpu.VMEM((NCHUNK, 128), jnp.int32),
            pltpu.VMEM((NCHUNK, 128), jnp.float32),
            pltpu.VMEM((CT * 4, DS), jnp.float32),
            pltpu.VMEM((CT * 4, DS), jnp.float32),
            pltpu.VMEM((CT, DS), jnp.float32),
            pltpu.SemaphoreType.DMA,
            pltpu.SemaphoreType.DMA,
        ],
    )
    out = pool(x, idx, wts)
    return out.reshape(B, N, DS)


# speedup vs baseline: 5.3320x; 1.5341x over previous
import jax
import jax.numpy as jnp
from jax import lax
from jax.experimental import pallas as pl
from jax.experimental.pallas import tpu as pltpu
from jax.experimental.pallas import tpu_sc as plsc

B, M, N, DA, DS = 16, 4096, 1024, 128, 128
R = B * M
T = B * N
NC, NS = 2, 16
NW = NC * NS
TPW = T // NW
CT = 64
NCHUNK = TPW // CT
PBLK = 4096


def _proj_body(a_ref, w_ref, x_ref):
    x_ref[...] = jnp.maximum(
        lax.dot_general(a_ref[...], w_ref[...], (((1,), (1,)), ((), ())),
                        preferred_element_type=jnp.float32),
        0.0)


def _sc_pool_body(x_hbm, lens_hbm, out_hbm,
                  lens_v, idx_v, w_v, bufA, bufB, out_v, semA, semB):
    wid = lax.axis_index("s") * NC + lax.axis_index("c")
    b = wid // 2
    h = wid % 2
    pltpu.sync_copy(lens_hbm.at[b], lens_v)

    acc = lens_v[pl.ds(0, 16)].astype(jnp.float32)
    for k in range(1, 32):
        acc = acc + lens_v[pl.ds(k * 16, 16)].astype(jnp.float32)
    mid = plsc.cumsum(acc)[15]
    carry = jnp.where(h == 1, mid, 0.0)

    boff = b * M
    half = h * TPW
    for k in range(32):
        v = lens_v[pl.ds(half + k * 16, 16)]
        vf = v.astype(jnp.float32)
        ends = plsc.cumsum(vf) + carry
        starts = (ends - vf).astype(jnp.int32)
        carry = ends[15]
        gi = starts + boff
        inv = 1.0 / jnp.maximum(vf, 1.0)
        for j in range(4):
            idx_v[j, pl.ds(k * 16, 16)] = jnp.minimum(gi + j, R - 1)
            w_v[j, pl.ds(k * 16, 16)] = jnp.where(v > j, inv, 0.0)

    bufs = (bufA, bufB)
    sems = (semA, semB)

    def fire(chunk, nbuf):
        buf, sem = bufs[nbuf], sems[nbuf]
        return [pltpu.async_copy(x_hbm.at[idx_v.at[j, pl.ds(chunk * CT, CT)]],
                                 buf.at[j], sem)
                for j in range(4)]

    def compute(chunk, nbuf):
        buf = bufs[nbuf]

        def group_body(tg, carry):
            wv0 = w_v[0, pl.ds(chunk * CT + tg * 16, 16)]
            wv1 = w_v[1, pl.ds(chunk * CT + tg * 16, 16)]
            wv2 = w_v[2, pl.ds(chunk * CT + tg * 16, 16)]
            wv3 = w_v[3, pl.ds(chunk * CT + tg * 16, 16)]
            for u in range(16):
                t = tg * 16 + u
                w0, w1, w2, w3 = wv0[u], wv1[u], wv2[u], wv3[u]
                for g in range(8):
                    s = pl.ds(g * 16, 16)
                    out_v[t, s] = (buf[0, t, s] * w0 + buf[1, t, s] * w1
                                   + buf[2, t, s] * w2 + buf[3, t, s] * w3)
            return carry

        lax.fori_loop(0, CT // 16, group_body, 0)
        pltpu.sync_copy(out_v, out_hbm.at[pl.ds(wid * TPW + chunk * CT, CT)])

    handles = [None, None]
    handles[0] = fire(0, 0)
    for chunk in range(NCHUNK):
        nb = chunk % 2
        if chunk + 1 < NCHUNK:
            handles[1 - nb] = fire(chunk + 1, 1 - nb)
        for hdl in handles[nb]:
            hdl.wait()
        compute(chunk, nb)


def kernel(atom_feats, atom_mask, molecule_atom_lens, W):
    del atom_mask
    a2 = atom_feats.reshape(R, DA)
    x = pl.pallas_call(
        _proj_body,
        grid=(R // PBLK,),
        in_specs=[pl.BlockSpec((PBLK, DA), lambda i: (i, 0)),
                  pl.BlockSpec((DS, DA), lambda i: (0, 0))],
        out_specs=pl.BlockSpec((PBLK, DS), lambda i: (i, 0)),
        out_shape=jax.ShapeDtypeStruct((R, DS), jnp.float32),
    )(a2, W)

    pool = pl.kernel(
        _sc_pool_body,
        out_type=jax.ShapeDtypeStruct((T, DS), jnp.float32),
        mesh=plsc.VectorSubcoreMesh(core_axis_name="c", subcore_axis_name="s"),
        compiler_params=pltpu.CompilerParams(needs_layout_passes=False),
        scratch_types=[
            pltpu.VMEM((N,), jnp.int32),
            pltpu.VMEM((4, TPW), jnp.int32),
            pltpu.VMEM((4, TPW), jnp.float32),
            pltpu.VMEM((4, CT, DS), jnp.float32),
            pltpu.VMEM((4, CT, DS), jnp.float32),
            pltpu.VMEM((CT, DS), jnp.float32),
            pltpu.SemaphoreType.DMA,
            pltpu.SemaphoreType.DMA,
        ],
    )
    out = pool(x, molecule_atom_lens)
    return out.reshape(B, N, DS)


# speedup vs baseline: 5.3396x; 1.0014x over previous
import jax
import jax.numpy as jnp
from jax import lax
from jax.experimental import pallas as pl
from jax.experimental.pallas import tpu as pltpu
from jax.experimental.pallas import tpu_sc as plsc

B, M, N, DA, DS = 16, 4096, 1024, 128, 128
R = B * M
T = B * N
NC, NS = 2, 16
NW = NC * NS
TPW = T // NW
CT = 64
NCHUNK = TPW // CT
PBLK = 4096


def _proj_body(a_ref, w_ref, x_ref):
    x_ref[...] = jnp.maximum(
        lax.dot_general(a_ref[...], w_ref[...], (((1,), (1,)), ((), ())),
                        preferred_element_type=jnp.float32),
        0.0)


def _sc_pool_body(x_hbm, lens_hbm, out_hbm,
                  lens_v, idx_v, w_v, bufA, bufB, out_v, semA, semB):
    wid = lax.axis_index("s") * NC + lax.axis_index("c")
    b = wid // 2
    h = wid % 2
    pltpu.sync_copy(lens_hbm.at[b], lens_v)

    acc = lens_v[pl.ds(0, 16)].astype(jnp.float32)
    for k in range(1, 32):
        acc = acc + lens_v[pl.ds(k * 16, 16)].astype(jnp.float32)
    mid = plsc.cumsum(acc)[15]
    carry = jnp.where(h == 1, mid, 0.0)

    boff = b * M
    half = h * TPW
    for k in range(32):
        v = lens_v[pl.ds(half + k * 16, 16)]
        vf = v.astype(jnp.float32)
        ends = plsc.cumsum(vf) + carry
        starts = (ends - vf).astype(jnp.int32)
        carry = ends[15]
        gi = starts + boff
        inv = 1.0 / jnp.maximum(vf, 1.0)
        for j in range(4):
            idx_v[j, pl.ds(k * 16, 16)] = jnp.minimum(gi + j, R - 1)
            w_v[j, pl.ds(k * 16, 16)] = jnp.where(v > j, inv, 0.0)

    bufs = (bufA, bufB)
    sems = (semA, semB)

    def fire(chunk, nbuf):
        buf, sem = bufs[nbuf], sems[nbuf]
        return [pltpu.async_copy(x_hbm.at[idx_v.at[j, pl.ds(chunk * CT, CT)]],
                                 buf.at[j], sem)
                for j in range(4)]

    def compute(chunk, nbuf):
        buf = bufs[nbuf]

        @plsc.parallel_loop(0, CT // 16, unroll=1)
        def group_body(tg):
            wv0 = w_v[0, pl.ds(chunk * CT + tg * 16, 16)]
            wv1 = w_v[1, pl.ds(chunk * CT + tg * 16, 16)]
            wv2 = w_v[2, pl.ds(chunk * CT + tg * 16, 16)]
            wv3 = w_v[3, pl.ds(chunk * CT + tg * 16, 16)]
            for u in range(16):
                t = tg * 16 + u
                w0, w1, w2, w3 = wv0[u], wv1[u], wv2[u], wv3[u]
                for g in range(8):
                    s = pl.ds(g * 16, 16)
                    out_v[t, s] = (buf[0, t, s] * w0 + buf[1, t, s] * w1
                                   + buf[2, t, s] * w2 + buf[3, t, s] * w3)

        pltpu.sync_copy(out_v, out_hbm.at[pl.ds(wid * TPW + chunk * CT, CT)])

    handles = [None, None]
    handles[0] = fire(0, 0)
    for chunk in range(NCHUNK):
        nb = chunk % 2
        if chunk + 1 < NCHUNK:
            handles[1 - nb] = fire(chunk + 1, 1 - nb)
        for hdl in handles[nb]:
            hdl.wait()
        compute(chunk, nb)


def kernel(atom_feats, atom_mask, molecule_atom_lens, W):
    del atom_mask
    a2 = atom_feats.reshape(R, DA)
    x = pl.pallas_call(
        _proj_body,
        grid=(R // PBLK,),
        in_specs=[pl.BlockSpec((PBLK, DA), lambda i: (i, 0)),
                  pl.BlockSpec((DS, DA), lambda i: (0, 0))],
        out_specs=pl.BlockSpec((PBLK, DS), lambda i: (i, 0)),
        out_shape=jax.ShapeDtypeStruct((R, DS), jnp.float32),
    )(a2, W)

    pool = pl.kernel(
        _sc_pool_body,
        out_type=jax.ShapeDtypeStruct((T, DS), jnp.float32),
        mesh=plsc.VectorSubcoreMesh(core_axis_name="c", subcore_axis_name="s"),
        compiler_params=pltpu.CompilerParams(needs_layout_passes=False),
        scratch_types=[
            pltpu.VMEM((N,), jnp.int32),
            pltpu.VMEM((4, TPW), jnp.int32),
            pltpu.VMEM((4, TPW), jnp.float32),
            pltpu.VMEM((4, CT, DS), jnp.float32),
            pltpu.VMEM((4, CT, DS), jnp.float32),
            pltpu.VMEM((CT, DS), jnp.float32),
            pltpu.SemaphoreType.DMA,
            pltpu.SemaphoreType.DMA,
        ],
    )
    out = pool(x, molecule_atom_lens)
    return out.reshape(B, N, DS)
